# baseline (speedup 1.0000x reference)
import functools

import jax
import jax.numpy as jnp
from jax import lax
from jax.experimental import pallas as pl
from jax.experimental.pallas import tpu as pltpu
from jax.experimental.pallas import tpu_sc as plsc

N = 10000
D = 128
NU = 5000
E = 160000
NC = 2
NS = 16
CHUNK = 128
CH = 79
EPT = CH * CHUNK
ROWS_PT = 320
ACC_ROWS = NS * ROWS_PT


def _prep_edges(rows, cols, vals):
    pad = NS * EPT - E
    rows = jnp.concatenate([rows, jnp.zeros((pad,), jnp.int32)])
    cols = jnp.concatenate([cols, jnp.zeros((pad,), jnp.int32)])
    vals = jnp.concatenate([vals, jnp.zeros((pad,), jnp.float32)])
    shp = (NS, CH, 1, CHUNK)
    idx = jnp.concatenate([rows.reshape(shp), cols.reshape(shp)], axis=2)
    return idx, vals.reshape((NS, CH, CHUNK))


_sc_mesh = plsc.VectorSubcoreMesh(
    core_axis_name="c", subcore_axis_name="s", num_cores=NC, num_subcores=NS)


@functools.partial(
    pl.kernel,
    out_type=jax.ShapeDtypeStruct((NC, ACC_ROWS, D), jnp.float32),
    mesh=_sc_mesh,
    scratch_types=[
        pltpu.VMEM((4, 2, CHUNK), jnp.int32),
        pltpu.VMEM((4, CHUNK), jnp.float32),
        pltpu.VMEM((3, CHUNK, D), jnp.float32),
        pltpu.VMEM_SHARED((ACC_ROWS, D), jnp.float32),
        pltpu.SemaphoreType.DMA,
        pltpu.SemaphoreType.DMA,
        pltpu.SemaphoreType.DMA,
        pltpu.SemaphoreType.DMA,
    ],
)
def _sc_spmm(old_hbm, idx_hbm, vals_hbm, zeros_hbm, out_hbm,
             ibuf, vbuf, gbuf, acc, isem, vsem, gsem, ssem):
    c = lax.axis_index("c")
    s = lax.axis_index("s")
    pltpu.sync_copy(zeros_hbm.at[pl.ds(s * ROWS_PT, ROWS_PT)],
                    acc.at[pl.ds(s * ROWS_PT, ROWS_PT)])
    plsc.subcore_barrier()

    def idx_cp(t, slot):
        return pltpu.make_async_copy(idx_hbm.at[c, s, t], ibuf.at[slot], isem)

    def val_cp(t, slot):
        return pltpu.make_async_copy(vals_hbm.at[c, s, t], vbuf.at[slot], vsem)

    def gather_cp(t3, t4):
        return pltpu.make_async_copy(
            old_hbm.at[ibuf.at[t4, 1]], gbuf.at[t3], gsem)

    idx_cp(0, 0).start()
    val_cp(0, 0).start()
    idx_cp(0, 0).wait()
    gather_cp(0, 0).start()
    val_cp(0, 0).wait()
    idx_cp(1, 1).start()
    val_cp(1, 1).start()

    def chunk_body(j, _):
        j3 = lax.rem(j, 3)
        j4 = lax.rem(j, 4)
        n3 = lax.rem(j + 1, 3)
        n4 = lax.rem(j + 1, 4)

        @pl.when(j + 1 < CH)
        def _():
            idx_cp(j + 1, n4).wait()
            val_cp(j + 1, n4).wait()

        @pl.when(j + 2 < CH)
        def _():
            idx_cp(j + 2, lax.rem(j + 2, 4)).start()
            val_cp(j + 2, lax.rem(j + 2, 4)).start()

        gather_cp(j3, j4).wait()

        @pl.when(j + 1 < CH)
        def _():
            gather_cp(n3, n4).start()

        def grp_body(g, _):
            vg = vbuf[j4, pl.ds(g * 16, 16)]
            for jj in range(16):
                v16 = vg.at[jnp.full((16,), jj, jnp.int32)].get(
                    mode="promise_in_bounds")
                i = g * 16 + jj
                for k in range(D // 16):
                    sl = pl.ds(k * 16, 16)
                    gbuf[j3, i, sl] = gbuf[j3, i, sl] * v16
            return 0

        lax.fori_loop(0, CHUNK // 16, grp_body, 0)

        @pl.when(j > 0)
        def _():
            p3 = lax.rem(j + 2, 3)
            p4 = lax.rem(j + 3, 4)
            pltpu.make_async_copy(
                gbuf.at[p3], acc.at[ibuf.at[p4, 0]], ssem).wait()

        pltpu.async_copy(gbuf.at[j3], acc.at[ibuf.at[j4, 0]], ssem, add=True)
        return 0

    lax.fori_loop(0, CH, chunk_body, 0)
    l3 = (CH - 1) % 3
    l4 = (CH - 1) % 4
    pltpu.make_async_copy(gbuf.at[l3], acc.at[ibuf.at[l4, 0]], ssem).wait()
    plsc.subcore_barrier()
    pltpu.sync_copy(acc.at[pl.ds(s * ROWS_PT, ROWS_PT)],
                    out_hbm.at[c, pl.ds(s * ROWS_PT, ROWS_PT)])


BR = 1000


def _dense_body(acc_ref, old_ref, ws_ref, wd_ref, out_ref):
    p = acc_ref[0]
    o = old_ref[...]
    x = (jnp.dot(p + o, ws_ref[0], preferred_element_type=jnp.float32)
         + jnp.dot(p * o, wd_ref[0], preferred_element_type=jnp.float32))
    out_ref[...] = jnp.maximum(x, 0.2 * x)


def _tc_dense(acc2, old, ws_stack, wd_stack):
    return pl.pallas_call(
        _dense_body,
        grid=(2, NU // BR),
        in_specs=[
            pl.BlockSpec((1, BR, D), lambda e, b: (e, b, 0)),
            pl.BlockSpec((BR, D), lambda e, b: (e * (NU // BR) + b, 0)),
            pl.BlockSpec((1, D, D), lambda e, b: (e, 0, 0)),
            pl.BlockSpec((1, D, D), lambda e, b: (e, 0, 0)),
        ],
        out_specs=pl.BlockSpec((BR, D), lambda e, b: (e * (NU // BR) + b, 0)),
        out_shape=jax.ShapeDtypeStruct((N, D), jnp.float32),
    )(acc2, old, ws_stack, wd_stack)


def kernel(initial_ebs, l_rows_user, l_cols_user, l_vals_user,
           li_rows_user, li_cols_user, li_vals_user,
           l_rows_item, l_cols_item, l_vals_item,
           li_rows_item, li_cols_item, li_vals_item,
           W_side_l0_user, W_dot_l0_user, W_side_l0_item, W_dot_l0_item,
           W_side_l1_user, W_dot_l1_user, W_side_l1_item, W_dot_l1_item,
           cluster_no, train_flag):
    iu, vu = _prep_edges(l_rows_user, l_cols_user, l_vals_user)
    ii, vi = _prep_edges(l_rows_item, l_cols_item, l_vals_item)
    idx_all = jnp.stack([iu, ii])
    vals_all = jnp.stack([vu, vi])
    zeros = jnp.zeros((ACC_ROWS, D), jnp.float32)

    ws = [jnp.stack([W_side_l0_user, W_side_l0_item]),
          jnp.stack([W_side_l1_user, W_side_l1_item])]
    wd = [jnp.stack([W_dot_l0_user, W_dot_l0_item]),
          jnp.stack([W_dot_l1_user, W_dot_l1_item])]

    old = initial_ebs
    outs = []
    for l in range(2):
        acc2 = _sc_spmm(old, idx_all, vals_all, zeros)
        old = _tc_dense(acc2, old, ws[l], wd[l])
        outs.append(old)
    return jnp.concatenate(outs, axis=0)
